# baseline (speedup 1.0000x reference)
import jax
import jax.numpy as jnp
from jax.experimental import pallas as pl
from jax.experimental.pallas import tpu as pltpu

_PAT_ROWS = 2048
_LANES = 128


def _mod_kernel(idx_ref, tab_ref, WT_ref, b_ref, out_hbm, patt_ref, sem):
    lane = idx_ref[0] % _LANES
    block = tab_ref[...]
    sel = (jax.lax.broadcasted_iota(jnp.int32, block.shape, 1) == lane)
    code = jnp.sum(jnp.where(sel, block, 0.0), axis=1)
    mods = jnp.dot(code, WT_ref[...], preferred_element_type=jnp.float32)
    mods = mods + b_ref[0, :]
    patt_ref[...] = jnp.broadcast_to(mods[None, :], patt_ref.shape)
    n = out_hbm.shape[0]
    copies = [
        pltpu.make_async_copy(
            patt_ref, out_hbm.at[pl.ds(k * _PAT_ROWS, _PAT_ROWS), :], sem)
        for k in range(n // _PAT_ROWS)
    ]
    for c in copies:
        c.start()
    for c in copies:
        c.wait()


def kernel(coords, patient_idx, emb_table, W, b):
    n = coords.shape[0]
    num_out, code_dim = W.shape
    idx = jnp.asarray(patient_idx, jnp.int32).reshape((1,))
    tabT = emb_table.T
    WT = W.T
    out = pl.pallas_call(
        _mod_kernel,
        grid_spec=pltpu.PrefetchScalarGridSpec(
            num_scalar_prefetch=1,
            grid=(1,),
            in_specs=[
                pl.BlockSpec((code_dim, _LANES), lambda i, idx_ref: (0, idx_ref[0] // _LANES)),
                pl.BlockSpec((code_dim, num_out), lambda i, idx_ref: (0, 0)),
                pl.BlockSpec((1, num_out), lambda i, idx_ref: (0, 0)),
            ],
            out_specs=pl.BlockSpec(memory_space=pl.ANY),
            scratch_shapes=[
                pltpu.VMEM((_PAT_ROWS, num_out), jnp.float32),
                pltpu.SemaphoreType.DMA,
            ],
        ),
        out_shape=jax.ShapeDtypeStruct((n, num_out), jnp.float32),
    )(idx, tabT, WT, b.reshape(1, num_out))
    return out


# speedup vs baseline: 1.0180x; 1.0180x over previous
import jax
import jax.numpy as jnp
from jax.experimental import pallas as pl
from jax.experimental.pallas import tpu as pltpu

_PAT_ROWS = 512
_LANES = 128


def _mod_kernel(idx_ref, tab_ref, WT_ref, b_ref, out_hbm, patt_ref, sem):
    lane = idx_ref[0] % _LANES
    block = tab_ref[...]
    sel = (jax.lax.broadcasted_iota(jnp.int32, block.shape, 1) == lane)
    code = jnp.sum(jnp.where(sel, block, 0.0), axis=1)
    mods = jnp.dot(code, WT_ref[...], preferred_element_type=jnp.float32)
    mods = mods + b_ref[0, :]
    patt_ref[...] = jnp.broadcast_to(mods[None, :], patt_ref.shape)
    n = out_hbm.shape[0]
    copies = [
        pltpu.make_async_copy(
            patt_ref, out_hbm.at[pl.ds(k * _PAT_ROWS, _PAT_ROWS), :], sem)
        for k in range(n // _PAT_ROWS)
    ]
    for c in copies:
        c.start()
    for c in copies:
        c.wait()


def kernel(coords, patient_idx, emb_table, W, b):
    n = coords.shape[0]
    num_out, code_dim = W.shape
    idx = jnp.asarray(patient_idx, jnp.int32).reshape((1,))
    tabT = emb_table.T
    WT = W.T
    out = pl.pallas_call(
        _mod_kernel,
        grid_spec=pltpu.PrefetchScalarGridSpec(
            num_scalar_prefetch=1,
            grid=(1,),
            in_specs=[
                pl.BlockSpec((code_dim, _LANES), lambda i, idx_ref: (0, idx_ref[0] // _LANES)),
                pl.BlockSpec((code_dim, num_out), lambda i, idx_ref: (0, 0)),
                pl.BlockSpec((1, num_out), lambda i, idx_ref: (0, 0)),
            ],
            out_specs=pl.BlockSpec(memory_space=pl.ANY),
            scratch_shapes=[
                pltpu.VMEM((_PAT_ROWS, num_out), jnp.float32),
                pltpu.SemaphoreType.DMA,
            ],
        ),
        out_shape=jax.ShapeDtypeStruct((n, num_out), jnp.float32),
    )(idx, tabT, WT, b.reshape(1, num_out))
    return out
